# baseline (speedup 1.0000x reference)
import functools

import jax
import jax.numpy as jnp
from jax import lax
from jax.experimental import pallas as pl
from jax.experimental.pallas import tpu as pltpu
from jax.experimental.pallas import tpu_sc as plsc

N = 10000
E = 320000
D = 128
NLAYERS = 5
NPAD = N + 16
SPLIT = 4608
ROWS0 = SPLIT // 16
ROWS1A = 336
ROWS1B = N - SPLIT - 15 * ROWS1A
ACCR = 5504
NW = 32
EPT = E // NW
CIN = 2000
NGRP = CIN // 16
CHK = 128
MAXCH = 80
CAP = MAXCH * CHK

_MESH = plsc.VectorSubcoreMesh(
    core_axis_name="c", subcore_axis_name="s", num_cores=2, num_subcores=16
)
_PARAMS = pltpu.CompilerParams(needs_layout_passes=False)

_i32 = jnp.int32
_f32 = jnp.float32


def _prep_body(src_hbm, dst_hbm, mask_hbm, scale_hbm, pert_hbm, w_hbm,
               srcO, dstO, wgtO, cntO, s0o, flago,
               s_in, d_in, m_in, sc_in, p_in, w_in,
               s_in2, d_in2, m_in2, sc_in2, p_in2, w_in2,
               sA, dA, wA, sB, dB, wB, stg_i, stg_f,
               ip_s, ip_d, ip_m, ip_c, ip_p, ip_w):
    c = lax.axis_index("c")
    s = lax.axis_index("s")
    r = c * 16 + s

    offA = _i32(0)
    offB = _i32(0)
    s0v = jnp.zeros((16,), _f32)
    flv = jnp.zeros((16,), _i32)

    inbufs = ((s_in, d_in, m_in, sc_in, p_in, w_in),
              (s_in2, d_in2, m_in2, sc_in2, p_in2, w_in2))
    insems = (ip_s, ip_d, ip_m, ip_c, ip_p, ip_w)

    def fire_in(ci, b):
        base = r * EPT + ci * CIN
        for ref, sem, src in zip(
                inbufs[b], insems,
                (src_hbm, dst_hbm, mask_hbm, scale_hbm, pert_hbm, w_hbm)):
            pltpu.async_copy(src.at[pl.ds(base, CIN)], ref, sem)

    def wait_in(b):
        for ref, sem, src in zip(
                inbufs[b], insems,
                (src_hbm, dst_hbm, mask_hbm, scale_hbm, pert_hbm, w_hbm)):
            pltpu.make_async_copy(src.at[pl.ds(0, CIN)], ref, sem).wait()

    fire_in(0, 0)
    for ci in range(EPT // CIN):
        b = ci % 2
        wait_in(b)
        if ci + 1 < EPT // CIN:
            fire_in(ci + 1, 1 - b)
        sb, db, mb, cb, pb, wb = inbufs[b]

        def grp(g, carry):
            offA, offB, s0v, flv = carry
            sl = pl.ds(g * 16, 16)
            mv = mb[sl]
            sv = sb[sl] * mv
            dv = db[sl] * mv
            wv = (wb[sl] * (1.0 - pb[sl]) + pb[sl]) * cb[sl]
            triv = jnp.logical_and(sv == 0, dv == 0)
            s0v = s0v + jnp.where(triv, wv, 0.0)
            nt = jnp.logical_not(triv)
            flv = flv | jnp.where(jnp.logical_and(nt, wv != 1.0), 1, 0)
            mA = jnp.logical_and(nt, dv < SPLIT)
            mB = jnp.logical_and(nt, dv >= SPLIT)
            csA = plsc.cumsum(mA.astype(_i32))
            csB = plsc.cumsum(mB.astype(_i32))
            idxA = csA - 1 + offA
            idxB = csB - 1 + offB
            rowA = lax.shift_right_logical(idxA, 7)
            colA = lax.bitwise_and(idxA, 127)
            rowB = lax.shift_right_logical(idxB, 7)
            colB = lax.bitwise_and(idxB, 127)
            plsc.store_scatter(sA, [rowA, colA], sv, mask=mA)
            plsc.store_scatter(dA, [rowA, colA], dv, mask=mA)
            plsc.store_scatter(wA, [rowA, colA], wv, mask=mA)
            plsc.store_scatter(sB, [rowB, colB], sv, mask=mB)
            plsc.store_scatter(dB, [rowB, colB], dv - SPLIT, mask=mB)
            plsc.store_scatter(wB, [rowB, colB], wv, mask=mB)
            nA = csA[15]
            nB = csB[15]
            return offA + nA, offB + nB, s0v, flv

        offA, offB, s0v, flv = lax.fori_loop(
            0, NGRP, grp, (offA, offB, s0v, flv))

    lane = lax.iota(_i32, 16)
    pad_src = lane + N
    pad_w = jnp.ones((16,), _f32)
    tgtA = ((offA + 4 * CHK - 1) // (4 * CHK)) * (4 * CHK)
    tgtB = ((offB + 4 * CHK - 1) // (4 * CHK)) * (4 * CHK)
    for g in range(4 * CHK // 16):
        pad_dst = lane + (g % 8) * 16
        idxA = lane + (offA + g * 16)
        idxB = lane + (offB + g * 16)
        mskA = idxA < tgtA
        mskB = idxB < tgtB
        rowA = lax.shift_right_logical(idxA, 7)
        colA = lax.bitwise_and(idxA, 127)
        rowB = lax.shift_right_logical(idxB, 7)
        colB = lax.bitwise_and(idxB, 127)
        plsc.store_scatter(sA, [rowA, colA], pad_src, mask=mskA)
        plsc.store_scatter(dA, [rowA, colA], pad_dst, mask=mskA)
        plsc.store_scatter(wA, [rowA, colA], pad_w, mask=mskA)
        plsc.store_scatter(sB, [rowB, colB], pad_src, mask=mskB)
        plsc.store_scatter(dB, [rowB, colB], pad_dst, mask=mskB)
        plsc.store_scatter(wB, [rowB, colB], pad_w, mask=mskB)

    pltpu.sync_copy(sA, srcO.at[0, r])
    pltpu.sync_copy(dA, dstO.at[0, r])
    pltpu.sync_copy(wA, wgtO.at[0, r])
    pltpu.sync_copy(sB, srcO.at[1, r])
    pltpu.sync_copy(dB, dstO.at[1, r])
    pltpu.sync_copy(wB, wgtO.at[1, r])

    zero16 = jnp.zeros((16,), _i32)
    stg_i[0] = zero16 + tgtA // CHK
    pltpu.sync_copy(stg_i, cntO.at[0, r])
    stg_i[0] = zero16 + tgtB // CHK
    pltpu.sync_copy(stg_i, cntO.at[1, r])

    stg_f[0] = jnp.zeros((16,), _f32) + jnp.sum(s0v)
    pltpu.sync_copy(stg_f, s0o.at[r])
    stg_i[0] = zero16 + jnp.max(flv)
    pltpu.sync_copy(stg_i, flago.at[r])


_prep = functools.partial(
    pl.kernel,
    out_type=(
        jax.ShapeDtypeStruct((2, NW, MAXCH, CHK), _i32),
        jax.ShapeDtypeStruct((2, NW, MAXCH, CHK), _i32),
        jax.ShapeDtypeStruct((2, NW, MAXCH, CHK), _f32),
        jax.ShapeDtypeStruct((2, NW, 1, 16), _i32),
        jax.ShapeDtypeStruct((NW, 1, 16), _f32),
        jax.ShapeDtypeStruct((NW, 1, 16), _i32),
    ),
    mesh=_MESH,
    compiler_params=_PARAMS,
    scratch_types=[
        pltpu.VMEM((CIN,), _i32),
        pltpu.VMEM((CIN,), _i32),
        pltpu.VMEM((CIN,), _i32),
        pltpu.VMEM((CIN,), _f32),
        pltpu.VMEM((CIN,), _f32),
        pltpu.VMEM((CIN,), _f32),
        pltpu.VMEM((CIN,), _i32),
        pltpu.VMEM((CIN,), _i32),
        pltpu.VMEM((CIN,), _i32),
        pltpu.VMEM((CIN,), _f32),
        pltpu.VMEM((CIN,), _f32),
        pltpu.VMEM((CIN,), _f32),
        pltpu.VMEM((MAXCH, CHK), _i32),
        pltpu.VMEM((MAXCH, CHK), _i32),
        pltpu.VMEM((MAXCH, CHK), _f32),
        pltpu.VMEM((MAXCH, CHK), _i32),
        pltpu.VMEM((MAXCH, CHK), _i32),
        pltpu.VMEM((MAXCH, CHK), _f32),
        pltpu.VMEM((1, 16), _i32),
        pltpu.VMEM((1, 16), _f32),
        pltpu.SemaphoreType.DMA,
        pltpu.SemaphoreType.DMA,
        pltpu.SemaphoreType.DMA,
        pltpu.SemaphoreType.DMA,
        pltpu.SemaphoreType.DMA,
        pltpu.SemaphoreType.DMA,
    ],
)(_prep_body)


def _clip_rows(clip_buf, nrows):
    lane = lax.iota(_i32, 16)
    zeros16 = jnp.zeros((16,), _i32)

    def crow(k, _):
        ksp = zeros16 + k
        for j in range(8):
            col = lane + j * 16
            v = plsc.load_gather(clip_buf, [ksp, col])
            v = jnp.minimum(jnp.maximum(v, 0.0), 2.0)
            plsc.store_scatter(clip_buf, [ksp, col], v)
        return 0

    lax.fori_loop(0, nrows, crow, 0)


def _layer_body(h_hbm, zeros_hbm, src_hbm, dst_hbm, wgt_hbm,
                cnt_hbm, s0i, flagi,
                h_out,
                acc, sidx, didx, wbuf, rows, cntv, flv, s0v, clip_buf,
                gs0, gs1, gs2, gs3, ss0, ss1, ss2, ss3, is_s, is_d, is_w,
                zsem, fsem, csem):
    c = lax.axis_index("c")
    s = lax.axis_index("s")
    lane = lax.iota(_i32, 16)
    zeros16 = jnp.zeros((16,), _i32)

    offz = pl.multiple_of(s * (ACCR // 16), 8)
    pltpu.async_copy(zeros_hbm.at[pl.ds(offz, ACCR // 16)],
                     acc.at[pl.ds(offz, ACCR // 16)], zsem)
    pltpu.async_copy(flagi, flv, fsem)
    pltpu.async_copy(cnt_hbm.at[c], cntv, csem)

    pltpu.make_async_copy(flagi, flv, fsem).wait()
    fv = flv[0, 0]
    for i in range(1, NW):
        fv = fv | flv[i, 0]
    flag_on = jnp.max(fv) > 0
    pltpu.make_async_copy(cnt_hbm.at[c], cntv, csem).wait()
    pltpu.make_async_copy(zeros_hbm.at[pl.ds(offz, ACCR // 16)],
                          acc.at[pl.ds(offz, ACCR // 16)], zsem).wait()

    plsc.subcore_barrier()

    gsems = (gs0, gs1, gs2, gs3)
    ssems = (ss0, ss1, ss2, ss3)

    def drain_scatter(b):
        pltpu.make_async_copy(rows.at[b], acc.at[didx.at[0, 0]],
                              ssems[b]).wait()

    def scale_chunk(b, j):
        jsp = zeros16 + j
        bsp = zeros16 + b

        def scale_k(k, _):
            ksp = zeros16 + k
            wsp = plsc.load_gather(wbuf, [jsp, ksp])
            for jj in range(8):
                col = lane + jj * 16
                v = plsc.load_gather(rows, [bsp, ksp, col])
                plsc.store_scatter(rows, [bsp, ksp, col], v * wsp)
            return 0

        lax.fori_loop(0, CHK, scale_k, 0)

    def one_region(i, _):
        reg = s + i * 16
        n_ch = cntv[reg, 0][0]
        n_sc = (n_ch + 7) // 8

        def fire_idx(q, qp):
            qb = pl.multiple_of(q * 8, 8)
            pltpu.async_copy(src_hbm.at[c, reg, pl.ds(qb, 8)],
                             sidx.at[qp], is_s)
            pltpu.async_copy(dst_hbm.at[c, reg, pl.ds(qb, 8)],
                             didx.at[qp], is_d)


        def wait_idx(qp):
            pltpu.make_async_copy(src_hbm.at[c, reg, pl.ds(0, 8)],
                                  sidx.at[qp], is_s).wait()
            pltpu.make_async_copy(dst_hbm.at[c, reg, pl.ds(0, 8)],
                                  didx.at[qp], is_d).wait()


        @pl.when(n_ch > 0)
        def _():
            fire_idx(0, 0)

        def sc_step(q, _):
            qp = lax.rem(q, 2)
            qo = 1 - qp
            for j in range(8):
                t = q * 8 + j
                if j >= 2:
                    @pl.when(t - 2 < n_ch)
                    def _():
                        drain_scatter((j + 2) % 4)
                else:
                    @pl.when(jnp.logical_and(t >= 2, t - 2 < n_ch))
                    def _():
                        drain_scatter((j + 2) % 4)
                if j == 0:
                    @pl.when(q == 0)
                    def _():
                        wait_idx(0)
                        pltpu.async_copy(h_hbm.at[sidx.at[0, 0]],
                                         rows.at[0], gsems[0])

                        @pl.when(n_ch > 1)
                        def _():
                            pltpu.async_copy(h_hbm.at[sidx.at[0, 1]],
                                             rows.at[1], gsems[1])

                    @pl.when((q + 1) * 8 < n_ch)
                    def _():
                        fire_idx(q + 1, qo)

                    @pl.when(flag_on)
                    def _():
                        pltpu.sync_copy(
                            wgt_hbm.at[c, reg,
                                       pl.ds(pl.multiple_of(q * 8, 8), 8)],
                            wbuf)
                if j < 6:
                    @pl.when(t + 2 < n_ch)
                    def _():
                        pltpu.async_copy(h_hbm.at[sidx.at[qp, j + 2]],
                                         rows.at[(j + 2) % 4],
                                         gsems[(j + 2) % 4])
                else:
                    @pl.when(t + 2 < n_ch)
                    def _():
                        if j == 6:
                            wait_idx(qo)
                        pltpu.async_copy(h_hbm.at[sidx.at[qo, (j + 2) % 8]],
                                         rows.at[(j + 2) % 4],
                                         gsems[(j + 2) % 4])
                @pl.when(t < n_ch)
                def _():
                    pltpu.make_async_copy(h_hbm.at[sidx.at[qp, j]],
                                          rows.at[j % 4],
                                          gsems[j % 4]).wait()

                    @pl.when(flag_on)
                    def _():
                        scale_chunk(j % 4, j)

                    pltpu.async_copy(rows.at[j % 4],
                                     acc.at[didx.at[qp, j]],
                                     ssems[j % 4], add=True)
            return 0

        lax.fori_loop(0, n_sc, sc_step, 0)

        @pl.when(jnp.logical_and(n_ch > 0, lax.rem(n_ch, 8) == 0))
        def _():
            drain_scatter(2)
            drain_scatter(3)
        return 0

    lax.fori_loop(0, 2, one_region, 0)

    plsc.subcore_barrier()

    is0 = c == 0
    nblk = jnp.where(is0, ROWS0 // 16,
                     jnp.where(s < 15, ROWS1A // 16, ROWS1B // 16))
    lbase = jnp.where(is0, s * ROWS0, s * ROWS1A)
    gbase = jnp.where(is0, s * ROWS0, SPLIT + s * ROWS1A)

    def clip16(bp):
        bpsp = zeros16 + bp

        def crow(k, _):
            ksp = zeros16 + k
            for j in range(8):
                col = lane + j * 16
                v = plsc.load_gather(clip_buf, [bpsp, ksp, col])
                v = jnp.minimum(jnp.maximum(v, 0.0), 2.0)
                plsc.store_scatter(clip_buf, [bpsp, ksp, col], v)
            return 0

        lax.fori_loop(0, 16, crow, 0)

    def fire_in_blk(b):
        o_l = pl.multiple_of(lbase + b * 16, 8)
        pltpu.async_copy(acc.at[pl.ds(o_l, 16)],
                         clip_buf.at[lax.rem(b, 2)], csem)

    @pl.when(nblk > 0)
    def _():
        fire_in_blk(0)

    def clip_blk(b, _):
        bp = lax.rem(b, 2)
        pltpu.make_async_copy(acc.at[pl.ds(0, 16)], clip_buf.at[bp],
                              csem).wait()

        @pl.when(b >= 1)
        def _():
            pltpu.make_async_copy(clip_buf.at[bp], h_out.at[pl.ds(0, 16)],
                                  fsem).wait()

        @pl.when(b + 1 < nblk)
        def _():
            fire_in_blk(b + 1)

        @pl.when(jnp.logical_and(jnp.logical_and(c == 0, s == 0), b == 0))
        def _():
            pltpu.sync_copy(s0i, s0v)
            tv = s0v[0, 0]
            for i in range(1, NW):
                tv = tv + s0v[i, 0]
            s0 = jnp.max(tv)
            pltpu.sync_copy(h_hbm.at[pl.ds(0, 8)], rows.at[0, pl.ds(0, 8)])
            for j in range(8):
                sl = pl.ds(j * 16, 16)
                clip_buf[0, 0, sl] = clip_buf[0, 0, sl] + s0 * rows[0, 0, sl]

        clip16(bp)
        o_g = pl.multiple_of(gbase + b * 16, 8)
        pltpu.async_copy(clip_buf.at[bp], h_out.at[pl.ds(o_g, 16)], fsem)
        return 0

    lax.fori_loop(0, nblk, clip_blk, 0)

    @pl.when(nblk > 0)
    def _():
        pltpu.make_async_copy(clip_buf.at[0], h_out.at[pl.ds(0, 16)],
                              fsem).wait()

    @pl.when(jnp.logical_and(c == 1, s == 15))
    def _():
        pltpu.sync_copy(zeros_hbm.at[pl.ds(0, 16)], h_out.at[pl.ds(N, 16)])


_layer = functools.partial(
    pl.kernel,
    out_type=jax.ShapeDtypeStruct((NPAD, D), _f32),
    mesh=_MESH,
    compiler_params=_PARAMS,
    scratch_types=[
        pltpu.VMEM_SHARED((ACCR, D), _f32),
        pltpu.VMEM((2, 8, CHK), _i32),
        pltpu.VMEM((2, 8, CHK), _i32),
        pltpu.VMEM((8, CHK), _f32),
        pltpu.VMEM((4, CHK, D), _f32),
        pltpu.VMEM((NW, 1, 16), _i32),
        pltpu.VMEM((NW, 1, 16), _i32),
        pltpu.VMEM((NW, 1, 16), _f32),
        pltpu.VMEM((2, 16, D), _f32),
        pltpu.SemaphoreType.DMA,
        pltpu.SemaphoreType.DMA,
        pltpu.SemaphoreType.DMA,
        pltpu.SemaphoreType.DMA,
        pltpu.SemaphoreType.DMA,
        pltpu.SemaphoreType.DMA,
        pltpu.SemaphoreType.DMA,
        pltpu.SemaphoreType.DMA,
        pltpu.SemaphoreType.DMA,
        pltpu.SemaphoreType.DMA,
        pltpu.SemaphoreType.DMA,
        pltpu.SemaphoreType.DMA,
        pltpu.SemaphoreType.DMA,
        pltpu.SemaphoreType.DMA,
    ],
)(_layer_body)


def kernel(x, edge_index, edge_mask, edge_scale, pert_mask, W):
    src_l, dst_l, wgt_l, cnt_l, s0p, flagp = _prep(
        edge_index[0], edge_index[1], edge_mask, edge_scale, pert_mask, W)
    zeros = jnp.zeros((ACCR, D), _f32)
    h = jnp.concatenate([x, jnp.zeros((NPAD - N, D), _f32)], axis=0)
    for _ in range(NLAYERS):
        h = _layer(h, zeros, src_l, dst_l, wgt_l, cnt_l, s0p, flagp)
    return h[:N]


# speedup vs baseline: 1.2364x; 1.2364x over previous
import functools

import jax
import jax.numpy as jnp
from jax import lax
from jax.experimental import pallas as pl
from jax.experimental.pallas import tpu as pltpu
from jax.experimental.pallas import tpu_sc as plsc

N = 10000
E = 320000
D = 128
NLAYERS = 5
NPAD = N + 16
SPLIT = 4864
ROWS0 = SPLIT // 16
ROWS1A = 320
ROWS1B = N - SPLIT - 15 * ROWS1A
ACCR = 5248
NW = 32
EPT = E // NW
CIN = 2000
NGRP = CIN // 16
CHK = 128
MAXCH = 80
CAP = MAXCH * CHK

_MESH = plsc.VectorSubcoreMesh(
    core_axis_name="c", subcore_axis_name="s", num_cores=2, num_subcores=16
)
_PARAMS = pltpu.CompilerParams(needs_layout_passes=False)

_i32 = jnp.int32
_f32 = jnp.float32


def _prep_body(src_hbm, dst_hbm, mask_hbm, scale_hbm, pert_hbm, w_hbm,
               srcO, dstO, wgtO, cntO, s0o, flago,
               s_in, d_in, m_in, sc_in, p_in, w_in,
               s_in2, d_in2, m_in2, sc_in2, p_in2, w_in2,
               sA, dA, wA, sB, dB, wB, stg_i, stg_f,
               ip_s, ip_d, ip_m, ip_c, ip_p, ip_w):
    c = lax.axis_index("c")
    s = lax.axis_index("s")
    r = c * 16 + s

    offA = _i32(0)
    offB = _i32(0)
    s0v = jnp.zeros((16,), _f32)
    flv = jnp.zeros((16,), _i32)

    inbufs = ((s_in, d_in, m_in, sc_in, p_in, w_in),
              (s_in2, d_in2, m_in2, sc_in2, p_in2, w_in2))
    insems = (ip_s, ip_d, ip_m, ip_c, ip_p, ip_w)

    def fire_in(ci, b):
        base = r * EPT + ci * CIN
        for ref, sem, src in zip(
                inbufs[b], insems,
                (src_hbm, dst_hbm, mask_hbm, scale_hbm, pert_hbm, w_hbm)):
            pltpu.async_copy(src.at[pl.ds(base, CIN)], ref, sem)

    def wait_in(b):
        for ref, sem, src in zip(
                inbufs[b], insems,
                (src_hbm, dst_hbm, mask_hbm, scale_hbm, pert_hbm, w_hbm)):
            pltpu.make_async_copy(src.at[pl.ds(0, CIN)], ref, sem).wait()

    fire_in(0, 0)
    for ci in range(EPT // CIN):
        b = ci % 2
        wait_in(b)
        if ci + 1 < EPT // CIN:
            fire_in(ci + 1, 1 - b)
        sb, db, mb, cb, pb, wb = inbufs[b]

        def grp(g, carry):
            offA, offB, s0v, flv = carry
            sl = pl.ds(g * 16, 16)
            mv = mb[sl]
            sv = sb[sl] * mv
            dv = db[sl] * mv
            wv = (wb[sl] * (1.0 - pb[sl]) + pb[sl]) * cb[sl]
            triv = jnp.logical_and(sv == 0, dv == 0)
            s0v = s0v + jnp.where(triv, wv, 0.0)
            nt = jnp.logical_not(triv)
            flv = flv | jnp.where(jnp.logical_and(nt, wv != 1.0), 1, 0)
            mA = jnp.logical_and(nt, dv < SPLIT)
            mB = jnp.logical_and(nt, dv >= SPLIT)
            csA = plsc.cumsum(mA.astype(_i32))
            csB = plsc.cumsum(mB.astype(_i32))
            idxA = csA - 1 + offA
            idxB = csB - 1 + offB
            rowA = lax.shift_right_logical(idxA, 7)
            colA = lax.bitwise_and(idxA, 127)
            rowB = lax.shift_right_logical(idxB, 7)
            colB = lax.bitwise_and(idxB, 127)
            plsc.store_scatter(sA, [rowA, colA], sv, mask=mA)
            plsc.store_scatter(dA, [rowA, colA], dv, mask=mA)
            plsc.store_scatter(wA, [rowA, colA], wv, mask=mA)
            plsc.store_scatter(sB, [rowB, colB], sv, mask=mB)
            plsc.store_scatter(dB, [rowB, colB], dv - SPLIT, mask=mB)
            plsc.store_scatter(wB, [rowB, colB], wv, mask=mB)
            nA = csA[15]
            nB = csB[15]
            return offA + nA, offB + nB, s0v, flv

        offA, offB, s0v, flv = lax.fori_loop(
            0, NGRP, grp, (offA, offB, s0v, flv))

    lane = lax.iota(_i32, 16)
    pad_src = lane + N
    pad_w = jnp.ones((16,), _f32)
    tgtA = ((offA + 4 * CHK - 1) // (4 * CHK)) * (4 * CHK)
    tgtB = ((offB + 4 * CHK - 1) // (4 * CHK)) * (4 * CHK)
    for g in range(4 * CHK // 16):
        pad_dst = lane + (g % 8) * 16
        idxA = lane + (offA + g * 16)
        idxB = lane + (offB + g * 16)
        mskA = idxA < tgtA
        mskB = idxB < tgtB
        rowA = lax.shift_right_logical(idxA, 7)
        colA = lax.bitwise_and(idxA, 127)
        rowB = lax.shift_right_logical(idxB, 7)
        colB = lax.bitwise_and(idxB, 127)
        plsc.store_scatter(sA, [rowA, colA], pad_src, mask=mskA)
        plsc.store_scatter(dA, [rowA, colA], pad_dst, mask=mskA)
        plsc.store_scatter(wA, [rowA, colA], pad_w, mask=mskA)
        plsc.store_scatter(sB, [rowB, colB], pad_src, mask=mskB)
        plsc.store_scatter(dB, [rowB, colB], pad_dst, mask=mskB)
        plsc.store_scatter(wB, [rowB, colB], pad_w, mask=mskB)

    pltpu.sync_copy(sA, srcO.at[0, r])
    pltpu.sync_copy(dA, dstO.at[0, r])
    pltpu.sync_copy(wA, wgtO.at[0, r])
    pltpu.sync_copy(sB, srcO.at[1, r])
    pltpu.sync_copy(dB, dstO.at[1, r])
    pltpu.sync_copy(wB, wgtO.at[1, r])

    zero16 = jnp.zeros((16,), _i32)
    stg_i[0] = zero16 + tgtA // CHK
    pltpu.sync_copy(stg_i, cntO.at[0, r])
    stg_i[0] = zero16 + tgtB // CHK
    pltpu.sync_copy(stg_i, cntO.at[1, r])

    stg_f[0] = jnp.zeros((16,), _f32) + jnp.sum(s0v)
    pltpu.sync_copy(stg_f, s0o.at[r])
    stg_i[0] = zero16 + jnp.max(flv)
    pltpu.sync_copy(stg_i, flago.at[r])


_prep = functools.partial(
    pl.kernel,
    out_type=(
        jax.ShapeDtypeStruct((2, NW, MAXCH, CHK), _i32),
        jax.ShapeDtypeStruct((2, NW, MAXCH, CHK), _i32),
        jax.ShapeDtypeStruct((2, NW, MAXCH, CHK), _f32),
        jax.ShapeDtypeStruct((2, NW, 1, 16), _i32),
        jax.ShapeDtypeStruct((NW, 1, 16), _f32),
        jax.ShapeDtypeStruct((NW, 1, 16), _i32),
    ),
    mesh=_MESH,
    compiler_params=_PARAMS,
    scratch_types=[
        pltpu.VMEM((CIN,), _i32),
        pltpu.VMEM((CIN,), _i32),
        pltpu.VMEM((CIN,), _i32),
        pltpu.VMEM((CIN,), _f32),
        pltpu.VMEM((CIN,), _f32),
        pltpu.VMEM((CIN,), _f32),
        pltpu.VMEM((CIN,), _i32),
        pltpu.VMEM((CIN,), _i32),
        pltpu.VMEM((CIN,), _i32),
        pltpu.VMEM((CIN,), _f32),
        pltpu.VMEM((CIN,), _f32),
        pltpu.VMEM((CIN,), _f32),
        pltpu.VMEM((MAXCH, CHK), _i32),
        pltpu.VMEM((MAXCH, CHK), _i32),
        pltpu.VMEM((MAXCH, CHK), _f32),
        pltpu.VMEM((MAXCH, CHK), _i32),
        pltpu.VMEM((MAXCH, CHK), _i32),
        pltpu.VMEM((MAXCH, CHK), _f32),
        pltpu.VMEM((1, 16), _i32),
        pltpu.VMEM((1, 16), _f32),
        pltpu.SemaphoreType.DMA,
        pltpu.SemaphoreType.DMA,
        pltpu.SemaphoreType.DMA,
        pltpu.SemaphoreType.DMA,
        pltpu.SemaphoreType.DMA,
        pltpu.SemaphoreType.DMA,
    ],
)(_prep_body)


def _clip_rows(clip_buf, nrows):
    lane = lax.iota(_i32, 16)
    zeros16 = jnp.zeros((16,), _i32)

    def crow(k, _):
        ksp = zeros16 + k
        for j in range(8):
            col = lane + j * 16
            v = plsc.load_gather(clip_buf, [ksp, col])
            v = jnp.minimum(jnp.maximum(v, 0.0), 2.0)
            plsc.store_scatter(clip_buf, [ksp, col], v)
        return 0

    lax.fori_loop(0, nrows, crow, 0)


def _layer_body(h_hbm, zeros_hbm, src_hbm, dst_hbm, wgt_hbm,
                cnt_hbm, s0i, flagi,
                h_out,
                acc, sidx, didx, wbuf, rows, cntv, flv, s0v, clip_buf,
                gs0, gs1, gs2, gs3, ss0, ss1, ss2, ss3, is_s, is_d, is_w,
                zsem, fsem, csem):
    c = lax.axis_index("c")
    s = lax.axis_index("s")
    lane = lax.iota(_i32, 16)
    zeros16 = jnp.zeros((16,), _i32)

    offz = pl.multiple_of(s * (ACCR // 16), 8)
    pltpu.async_copy(zeros_hbm.at[pl.ds(offz, ACCR // 16)],
                     acc.at[pl.ds(offz, ACCR // 16)], zsem)
    pltpu.async_copy(flagi, flv, fsem)
    pltpu.async_copy(cnt_hbm.at[c], cntv, csem)

    pltpu.make_async_copy(flagi, flv, fsem).wait()
    fv = flv[0, 0]
    for i in range(1, NW):
        fv = fv | flv[i, 0]
    flag_on = jnp.max(fv) > 0
    pltpu.make_async_copy(cnt_hbm.at[c], cntv, csem).wait()
    pltpu.make_async_copy(zeros_hbm.at[pl.ds(offz, ACCR // 16)],
                          acc.at[pl.ds(offz, ACCR // 16)], zsem).wait()

    plsc.subcore_barrier()

    gsems = (gs0, gs1, gs2, gs3)
    ssems = (ss0, ss1, ss2, ss3)

    def drain_scatter(b):
        pltpu.make_async_copy(rows.at[b], acc.at[didx.at[0, 0]],
                              ssems[b]).wait()

    def scale_chunk(b, j):
        jsp = zeros16 + j
        bsp = zeros16 + b

        def scale_k(k, _):
            ksp = zeros16 + k
            wsp = plsc.load_gather(wbuf, [jsp, ksp])
            for jj in range(8):
                col = lane + jj * 16
                v = plsc.load_gather(rows, [bsp, ksp, col])
                plsc.store_scatter(rows, [bsp, ksp, col], v * wsp)
            return 0

        lax.fori_loop(0, CHK, scale_k, 0)

    def one_region(i, _):
        reg = s + i * 16
        n_ch = cntv[reg, 0][0]
        n_sc = (n_ch + 7) // 8

        def fire_idx(q, qp):
            qb = pl.multiple_of(q * 8, 8)
            pltpu.async_copy(src_hbm.at[c, reg, pl.ds(qb, 8)],
                             sidx.at[qp], is_s)
            pltpu.async_copy(dst_hbm.at[c, reg, pl.ds(qb, 8)],
                             didx.at[qp], is_d)


        def wait_idx(qp):
            pltpu.make_async_copy(src_hbm.at[c, reg, pl.ds(0, 8)],
                                  sidx.at[qp], is_s).wait()
            pltpu.make_async_copy(dst_hbm.at[c, reg, pl.ds(0, 8)],
                                  didx.at[qp], is_d).wait()


        @pl.when(n_ch > 0)
        def _():
            fire_idx(0, 0)

        def sc_step(q, _):
            qp = lax.rem(q, 2)
            qo = 1 - qp
            for j in range(8):
                t = q * 8 + j
                if j >= 2:
                    @pl.when(t - 2 < n_ch)
                    def _():
                        drain_scatter((j + 2) % 4)
                else:
                    @pl.when(jnp.logical_and(t >= 2, t - 2 < n_ch))
                    def _():
                        drain_scatter((j + 2) % 4)
                if j == 0:
                    @pl.when(q == 0)
                    def _():
                        wait_idx(0)
                        pltpu.async_copy(h_hbm.at[sidx.at[0, 0]],
                                         rows.at[0], gsems[0])

                        @pl.when(n_ch > 1)
                        def _():
                            pltpu.async_copy(h_hbm.at[sidx.at[0, 1]],
                                             rows.at[1], gsems[1])

                    @pl.when((q + 1) * 8 < n_ch)
                    def _():
                        fire_idx(q + 1, qo)

                    @pl.when(flag_on)
                    def _():
                        pltpu.sync_copy(
                            wgt_hbm.at[c, reg,
                                       pl.ds(pl.multiple_of(q * 8, 8), 8)],
                            wbuf)
                if j < 6:
                    @pl.when(t + 2 < n_ch)
                    def _():
                        pltpu.async_copy(h_hbm.at[sidx.at[qp, j + 2]],
                                         rows.at[(j + 2) % 4],
                                         gsems[(j + 2) % 4])
                else:
                    @pl.when(t + 2 < n_ch)
                    def _():
                        if j == 6:
                            wait_idx(qo)
                        pltpu.async_copy(h_hbm.at[sidx.at[qo, (j + 2) % 8]],
                                         rows.at[(j + 2) % 4],
                                         gsems[(j + 2) % 4])
                @pl.when(t < n_ch)
                def _():
                    pltpu.make_async_copy(h_hbm.at[sidx.at[qp, j]],
                                          rows.at[j % 4],
                                          gsems[j % 4]).wait()

                    @pl.when(flag_on)
                    def _():
                        scale_chunk(j % 4, j)

                    pltpu.async_copy(rows.at[j % 4],
                                     acc.at[didx.at[qp, j]],
                                     ssems[j % 4], add=True)
            return 0

        lax.fori_loop(0, n_sc, sc_step, 0)

        @pl.when(jnp.logical_and(n_ch > 0, lax.rem(n_ch, 8) == 0))
        def _():
            drain_scatter(2)
            drain_scatter(3)
        return 0

    lax.fori_loop(0, 2, one_region, 0)

    plsc.subcore_barrier()

    is0 = c == 0
    nblk = jnp.where(is0, ROWS0 // 16,
                     jnp.where(s < 15, ROWS1A // 16, ROWS1B // 16))
    lbase = jnp.where(is0, s * ROWS0, s * ROWS1A)
    gbase = jnp.where(is0, s * ROWS0, SPLIT + s * ROWS1A)

    def clip16(bp):
        bpsp = zeros16 + bp

        def crow(k, _):
            ksp = zeros16 + k
            for j in range(8):
                col = lane + j * 16
                v = plsc.load_gather(clip_buf, [bpsp, ksp, col])
                v = jnp.minimum(jnp.maximum(v, 0.0), 2.0)
                plsc.store_scatter(clip_buf, [bpsp, ksp, col], v)
            return 0

        lax.fori_loop(0, 16, crow, 0)

    def fire_in_blk(b):
        o_l = pl.multiple_of(lbase + b * 16, 8)
        pltpu.async_copy(acc.at[pl.ds(o_l, 16)],
                         clip_buf.at[lax.rem(b, 2)], csem)

    @pl.when(nblk > 0)
    def _():
        fire_in_blk(0)

    def clip_blk(b, _):
        bp = lax.rem(b, 2)
        pltpu.make_async_copy(acc.at[pl.ds(0, 16)], clip_buf.at[bp],
                              csem).wait()

        @pl.when(b >= 1)
        def _():
            pltpu.make_async_copy(clip_buf.at[bp], h_out.at[pl.ds(0, 16)],
                                  fsem).wait()

        @pl.when(b + 1 < nblk)
        def _():
            fire_in_blk(b + 1)

        @pl.when(jnp.logical_and(jnp.logical_and(c == 0, s == 0), b == 0))
        def _():
            pltpu.sync_copy(s0i, s0v)
            tv = s0v[0, 0]
            for i in range(1, NW):
                tv = tv + s0v[i, 0]
            s0 = jnp.max(tv)
            pltpu.sync_copy(h_hbm.at[pl.ds(0, 8)], rows.at[0, pl.ds(0, 8)])
            for j in range(8):
                sl = pl.ds(j * 16, 16)
                clip_buf[0, 0, sl] = clip_buf[0, 0, sl] + s0 * rows[0, 0, sl]

        clip16(bp)
        o_g = pl.multiple_of(gbase + b * 16, 8)
        pltpu.async_copy(clip_buf.at[bp], h_out.at[pl.ds(o_g, 16)], fsem)
        return 0

    lax.fori_loop(0, nblk, clip_blk, 0)

    @pl.when(nblk > 0)
    def _():
        pltpu.make_async_copy(clip_buf.at[0], h_out.at[pl.ds(0, 16)],
                              fsem).wait()

    @pl.when(jnp.logical_and(c == 1, s == 15))
    def _():
        pltpu.sync_copy(zeros_hbm.at[pl.ds(0, 16)], h_out.at[pl.ds(N, 16)])


_layer = functools.partial(
    pl.kernel,
    out_type=jax.ShapeDtypeStruct((NPAD, D), _f32),
    mesh=_MESH,
    compiler_params=_PARAMS,
    scratch_types=[
        pltpu.VMEM_SHARED((ACCR, D), _f32),
        pltpu.VMEM((2, 8, CHK), _i32),
        pltpu.VMEM((2, 8, CHK), _i32),
        pltpu.VMEM((8, CHK), _f32),
        pltpu.VMEM((4, CHK, D), _f32),
        pltpu.VMEM((NW, 1, 16), _i32),
        pltpu.VMEM((NW, 1, 16), _i32),
        pltpu.VMEM((NW, 1, 16), _f32),
        pltpu.VMEM((2, 16, D), _f32),
        pltpu.SemaphoreType.DMA,
        pltpu.SemaphoreType.DMA,
        pltpu.SemaphoreType.DMA,
        pltpu.SemaphoreType.DMA,
        pltpu.SemaphoreType.DMA,
        pltpu.SemaphoreType.DMA,
        pltpu.SemaphoreType.DMA,
        pltpu.SemaphoreType.DMA,
        pltpu.SemaphoreType.DMA,
        pltpu.SemaphoreType.DMA,
        pltpu.SemaphoreType.DMA,
        pltpu.SemaphoreType.DMA,
        pltpu.SemaphoreType.DMA,
        pltpu.SemaphoreType.DMA,
    ],
)(_layer_body)


def kernel(x, edge_index, edge_mask, edge_scale, pert_mask, W):
    src_l, dst_l, wgt_l, cnt_l, s0p, flagp = _prep(
        edge_index[0], edge_index[1], edge_mask, edge_scale, pert_mask, W)
    zeros = jnp.zeros((ACCR, D), _f32)
    h = jnp.concatenate([x, jnp.zeros((NPAD - N, D), _f32)], axis=0)
    for _ in range(NLAYERS):
        h = _layer(h, zeros, src_l, dst_l, wgt_l, cnt_l, s0p, flagp)
    return h[:N]
